# baseline (speedup 1.0000x reference)
import functools

import jax
import jax.numpy as jnp
from jax import lax
from jax.experimental import pallas as pl
from jax.experimental.pallas import tpu as pltpu
from jax.experimental.pallas import tpu_sc as plsc

NUM_ENT = 1000000
NUM_REL = 1000
DIM = 64
BATCH = 16384

NC = 2
NS = 16
NW = NC * NS
B_PER_W = BATCH // NW
CHUNK = 128
NCHUNK = B_PER_W // CHUNK


def _transe_body(subj_hbm, obj_hbm, rel_hbm, ent_hbm, reltab_hbm, out_hbm,
                 idx_s, idx_o, idx_r, sub_v, obj_v, rel_v, score_v, sem):
    wid = lax.axis_index("s") * NC + lax.axis_index("c")

    pltpu.sync_copy(subj_hbm.at[wid], idx_s)
    pltpu.sync_copy(obj_hbm.at[wid], idx_o)
    pltpu.sync_copy(rel_hbm.at[wid], idx_r)

    copies = []
    for j in range(NCHUNK):
        sl = pl.ds(j * CHUNK, CHUNK)
        copies.append(pltpu.async_copy(ent_hbm.at[idx_s.at[j]], sub_v.at[sl], sem))
        copies.append(pltpu.async_copy(ent_hbm.at[idx_o.at[j]], obj_v.at[sl], sem))
        copies.append(pltpu.async_copy(reltab_hbm.at[idx_r.at[j]], rel_v.at[sl], sem))
    for c in copies:
        c.wait()

    lane = lax.iota(jnp.int32, 16)

    def group(g, _):
        row = g * 16 + lane
        acc = jnp.zeros((16,), jnp.float32)
        for d in range(DIM):
            col = jnp.full((16,), d, jnp.int32)
            s = plsc.load_gather(sub_v, [row, col])
            r = plsc.load_gather(rel_v, [row, col])
            o = plsc.load_gather(obj_v, [row, col])
            diff = s + r - o
            acc = acc + diff * diff
        score_v[pl.ds(g * 16, 16)] = acc
        return 0

    lax.fori_loop(0, B_PER_W // 16, group, 0)

    pltpu.sync_copy(score_v, out_hbm.at[wid])


@jax.jit
def _transe(subjects, objects, relations, ent_embedding, rel_embedding):
    mesh = plsc.VectorSubcoreMesh(core_axis_name="c", subcore_axis_name="s")
    kern = pl.kernel(
        _transe_body,
        out_type=jax.ShapeDtypeStruct((NW, B_PER_W), jnp.float32),
        mesh=mesh,
        scratch_types=[
            pltpu.VMEM((NCHUNK, CHUNK), jnp.int32),
            pltpu.VMEM((NCHUNK, CHUNK), jnp.int32),
            pltpu.VMEM((NCHUNK, CHUNK), jnp.int32),
            pltpu.VMEM((B_PER_W, DIM), jnp.float32),
            pltpu.VMEM((B_PER_W, DIM), jnp.float32),
            pltpu.VMEM((B_PER_W, DIM), jnp.float32),
            pltpu.VMEM((B_PER_W,), jnp.float32),
            pltpu.SemaphoreType.DMA,
        ],
        compiler_params=pltpu.CompilerParams(
            needs_layout_passes=False, use_tc_tiling_on_sc=False),
    )
    subj = subjects.astype(jnp.int32).reshape(NW, NCHUNK, CHUNK)
    obj = objects.astype(jnp.int32).reshape(NW, NCHUNK, CHUNK)
    rel = relations.astype(jnp.int32).reshape(NW, NCHUNK, CHUNK)
    out = kern(subj, obj, rel, ent_embedding, rel_embedding)
    return out.reshape(BATCH, 1)


def kernel(subjects, objects, relations, ent_embedding, rel_embedding):
    return _transe(subjects, objects, relations, ent_embedding, rel_embedding)


# speedup vs baseline: 1.7197x; 1.7197x over previous
import functools

import jax
import jax.numpy as jnp
from jax import lax
from jax.experimental import pallas as pl
from jax.experimental.pallas import tpu as pltpu
from jax.experimental.pallas import tpu_sc as plsc

NUM_ENT = 1000000
NUM_REL = 1000
DIM = 64
BATCH = 16384

NC = 2
NS = 16
NW = NC * NS
B_PER_W = BATCH // NW
GRP = 16
NGRP = B_PER_W // GRP
NBUF = 4


def _issue_group(g, bi, idx_s, idx_o, idx_r, ent_hbm, reltab_hbm,
                 sub_b, obj_b, rel_b, sems):
    vs = idx_s[pl.ds(g * GRP, GRP)]
    vo = idx_o[pl.ds(g * GRP, GRP)]
    vr = idx_r[pl.ds(g * GRP, GRP)]
    for l in range(GRP):
        row = pl.ds(l, 1)
        pltpu.async_copy(ent_hbm.at[pl.ds(vs[l], 1)], sub_b.at[bi, row], sems[bi][0])
        pltpu.async_copy(ent_hbm.at[pl.ds(vo[l], 1)], obj_b.at[bi, row], sems[bi][1])
        pltpu.async_copy(reltab_hbm.at[pl.ds(vr[l], 1)], rel_b.at[bi, row], sems[bi][2])


def _transe_body(subj_hbm, obj_hbm, rel_hbm, ent_hbm, reltab_hbm, out_hbm,
                 idx_s, idx_o, idx_r, sub_b, obj_b, rel_b, score_v,
                 s0, s1, s2, s3, s4, s5, s6, s7, s8, s9, s10, s11):
    wid = lax.axis_index("s") * NC + lax.axis_index("c")
    sems = [(s0, s1, s2), (s3, s4, s5), (s6, s7, s8), (s9, s10, s11)]

    pltpu.sync_copy(subj_hbm.at[wid], idx_s)
    pltpu.sync_copy(obj_hbm.at[wid], idx_o)
    pltpu.sync_copy(rel_hbm.at[wid], idx_r)

    lane = lax.iota(jnp.int32, GRP)

    for g in range(NBUF - 1):
        _issue_group(g, g, idx_s, idx_o, idx_r, ent_hbm, reltab_hbm,
                     sub_b, obj_b, rel_b, sems)

    def outer(h, _):
        for p in range(NBUF):
            g = h * NBUF + p

            pltpu.make_async_copy(ent_hbm.at[pl.ds(0, GRP)], sub_b.at[p], sems[p][0]).wait()
            pltpu.make_async_copy(ent_hbm.at[pl.ds(0, GRP)], obj_b.at[p], sems[p][1]).wait()
            pltpu.make_async_copy(reltab_hbm.at[pl.ds(0, GRP)], rel_b.at[p], sems[p][2]).wait()

            nb = (p + NBUF - 1) % NBUF

            @pl.when(g + NBUF - 1 < NGRP)
            def _():
                _issue_group(g + NBUF - 1, nb, idx_s, idx_o, idx_r,
                             ent_hbm, reltab_hbm, sub_b, obj_b, rel_b, sems)

            out = jnp.zeros((GRP,), jnp.float32)
            for l in range(GRP):
                acc = jnp.zeros((16,), jnp.float32)
                for c in range(DIM // 16):
                    col = pl.ds(c * 16, 16)
                    d = sub_b[p, l, col] + rel_b[p, l, col] - obj_b[p, l, col]
                    acc = acc + d * d
                s = lax.reduce_sum(acc, axes=(0,))
                out = jnp.where(lane == l, s, out)
            score_v[pl.ds(g * GRP, GRP)] = out
        return 0

    lax.fori_loop(0, NGRP // NBUF, outer, 0)

    pltpu.sync_copy(score_v, out_hbm.at[wid])


@jax.jit
def _transe(subjects, objects, relations, ent_embedding, rel_embedding):
    mesh = plsc.VectorSubcoreMesh(core_axis_name="c", subcore_axis_name="s")
    kern = pl.kernel(
        _transe_body,
        out_type=jax.ShapeDtypeStruct((NW, B_PER_W), jnp.float32),
        mesh=mesh,
        scratch_types=[
            pltpu.VMEM((B_PER_W,), jnp.int32),
            pltpu.VMEM((B_PER_W,), jnp.int32),
            pltpu.VMEM((B_PER_W,), jnp.int32),
            pltpu.VMEM((NBUF, GRP, DIM), jnp.float32),
            pltpu.VMEM((NBUF, GRP, DIM), jnp.float32),
            pltpu.VMEM((NBUF, GRP, DIM), jnp.float32),
            pltpu.VMEM((B_PER_W,), jnp.float32),
        ] + [pltpu.SemaphoreType.DMA] * 12,
        compiler_params=pltpu.CompilerParams(
            needs_layout_passes=False, use_tc_tiling_on_sc=True),
    )
    subj = subjects.astype(jnp.int32).reshape(NW, B_PER_W)
    obj = objects.astype(jnp.int32).reshape(NW, B_PER_W)
    rel = relations.astype(jnp.int32).reshape(NW, B_PER_W)
    out = kern(subj, obj, rel, ent_embedding, rel_embedding)
    return out.reshape(BATCH, 1)


def kernel(subjects, objects, relations, ent_embedding, rel_embedding):
    return _transe(subjects, objects, relations, ent_embedding, rel_embedding)
